# baseline (speedup 1.0000x reference)
import functools

import numpy as np
import jax
import jax.numpy as jnp
from jax import lax
from jax.experimental import pallas as pl
from jax.experimental.pallas import tpu as pltpu
from jax.experimental.pallas import tpu_sc as plsc

_N_LEVELS = 16
_TS = 2 ** 19
_MASK = _TS - 1
_PRIME = np.int32(np.uint32(2654435761).astype(np.int64) - (1 << 32))
_PAIRS = ((0, 1), (0, 2), (1, 2))
_NCOMBO = _N_LEVELS * 3
_NRLVL = 10
_NRC = 3 * _NRLVL

_B = 1048576
_NT = 32
_PPT = _B // _NT
_C = 256
_G = _C // 16
_NCHUNK = _PPT // _C
_BSEG = 2048


def _resolutions():
    growth = np.exp((np.log(512.0) - np.log(16.0)) / (_N_LEVELS - 1))
    return [int(np.floor(16.0 * (growth ** i))) for i in range(_N_LEVELS)]


def _combo_consts():
    invg = np.zeros((_NCOMBO, 16), np.float32)
    icon = np.zeros((3, _NCOMBO, 16), np.int32)
    for i, res in enumerate(_resolutions()):
        for j, (a, b) in enumerate(_PAIRS):
            c = 3 * i + j
            invg[c, :] = np.float32(res / 2.0)
            icon[0, c, :] = j * _N_LEVELS + i
            icon[1, c, :] = a
            icon[2, c, :] = b
    return invg.reshape(-1), icon.reshape(-1)


def _subtable_consts():
    iconr = np.zeros((3, _NRC, 16), np.int32)
    bidx_parts = []
    plan = []
    sb = 0
    boff = 0
    for i, res in enumerate(_resolutions()[:_NRLVL]):
        lo = res // 2
        W = res - lo + 1
        e = np.arange(W * W, dtype=np.int64)
        c0 = lo + e // W
        c1 = lo + e % W
        h = ((c0 ^ (c1 * 2654435761)) & _MASK).astype(np.int64)
        seg = np.empty(2 * W * W, dtype=np.int32)
        seg[0::2] = 2 * h
        seg[1::2] = 2 * h + 1
        pad = (-len(seg)) % 8
        if pad:
            seg = np.concatenate([seg, np.zeros(pad, np.int32)])
        for j in range(3):
            rc = 3 * i + j
            iconr[0, rc, :] = lo
            iconr[1, rc, :] = 2 * W
            iconr[2, rc, :] = sb
            bidx_parts.append(seg)
            for s in range(0, len(seg), _BSEG):
                ln = min(_BSEG, len(seg) - s)
                plan.append((j * _N_LEVELS + i, boff + s, sb + s, ln))
            sb += len(seg)
            boff += len(seg)
    return iconr.reshape(-1), np.concatenate(bidx_parts), plan, sb


_INVG_NP, _ICON_NP = _combo_consts()
_ICONR_NP, _BIDX_NP, _BUILD_PLAN, _STB_WORDS = _subtable_consts()


def _tile_body(xf, tflat, invg, icon, iconr, bidx, out,
               x_v, out_v, stb_v, idx_vs, rows_vs, w_vs,
               invg_v, icon_v, iconr_v, sems):
    wid = lax.axis_index("s") * 2 + lax.axis_index("c")
    iota = lax.iota(jnp.int32, 16)
    iota3 = iota * 3
    pltpu.sync_copy(invg, invg_v)
    pltpu.sync_copy(icon, icon_v)
    pltpu.sync_copy(iconr, iconr_v)

    for row, bo, so, ln in _BUILD_PLAN:
        pltpu.sync_copy(bidx.at[pl.ds(bo, ln)], idx_vs[0].at[pl.ds(0, ln)])
        pltpu.async_copy(
            tflat.at[row].at[idx_vs[0].at[pl.ds(0, ln)]],
            stb_v.at[pl.ds(so, ln)], sems.at[0])
        pltpu.make_async_copy(
            tflat.at[row].at[idx_vs[0].at[pl.ds(0, ln)]],
            stb_v.at[pl.ds(so, ln)], sems.at[0]).wait()

    def loadx(g, conda, condb):
        o = g * 16
        rows = iota + o
        x0 = plsc.load_gather(x_v, [rows, iota * 0])
        x1 = plsc.load_gather(x_v, [rows, iota * 0 + 1])
        x2 = plsc.load_gather(x_v, [rows, iota * 0 + 2])
        xa = jnp.where(conda, x0, x1)
        xb = jnp.where(condb, x1, x2)
        return xa, xb

    def combo_consts(c):
        igv = invg_v[pl.ds(c * 16, 16)]
        av = icon_v[pl.ds((_NCOMBO + c) * 16, 16)]
        bv = icon_v[pl.ds((2 * _NCOMBO + c) * 16, 16)]
        return igv, av == 0, bv == 1

    def resident(rc):
        igv, conda, condb = combo_consts(rc)
        lo = iconr_v[pl.ds(rc * 16, 16)]
        w2 = iconr_v[pl.ds((_NRC + rc) * 16, 16)]
        sb = iconr_v[pl.ds((2 * _NRC + rc) * 16, 16)]
        iam = lo + lax.shift_right_logical(w2, 1) - 2
        colbase = 2 * rc

        def grp(g, _):
            o = g * 16
            xa, xb = loadx(g, conda, condb)
            ta = (xa + 1.0) * igv
            tb = (xb + 1.0) * igv
            ia0 = jnp.minimum(ta.astype(jnp.int32), iam)
            ib0 = jnp.minimum(tb.astype(jnp.int32), iam)
            w0 = ta - ia0.astype(jnp.float32)
            w1 = tb - ib0.astype(jnp.float32)
            u0 = 1.0 - w0
            u1 = 1.0 - w1
            widx = sb + (ia0 - lo) * w2 + lax.shift_left(ib0 - lo, 1)
            g10 = widx + w2
            obase = (g * 16 + iota) * 96 + colbase
            for f in range(2):
                e00 = plsc.load_gather(stb_v, [widx + f])
                e01 = plsc.load_gather(stb_v, [widx + (2 + f)])
                e10 = plsc.load_gather(stb_v, [g10 + f])
                e11 = plsc.load_gather(stb_v, [g10 + (2 + f)])
                v0 = e00 * u0 + e10 * w0
                v1 = e01 * u0 + e11 * w0
                plsc.store_scatter(out_v, [obase + f], v0 * u1 + v1 * w1)
            return 0

        lax.fori_loop(0, _G, grp, 0)

    def fire(c, idx_v, w_v, rows_v, sem_i):
        igv, conda, condb = combo_consts(c)
        cc = icon_v[pl.ds(c * 16, 16)]

        def grp(g, _):
            o = g * 16
            xa, xb = loadx(g, conda, condb)
            ta = (xa + 1.0) * igv
            tb = (xb + 1.0) * igv
            ia0 = ta.astype(jnp.int32)
            ib0 = tb.astype(jnp.int32)
            w_v[pl.ds(o, 16)] = ta - ia0.astype(jnp.float32)
            w_v[pl.ds(_C + o, 16)] = tb - ib0.astype(jnp.float32)
            ia1 = ia0 + 1
            hb0 = ib0 * _PRIME
            hb1 = hb0 + _PRIME
            for k, (pa, pb) in enumerate(((ia0, hb0), (ia0, hb1),
                                          (ia1, hb0), (ia1, hb1))):
                f0 = ((pa ^ pb) & _MASK) * 2
                idx_v[pl.ds(k * _C + o, 16)] = f0
                idx_v[pl.ds(4 * _C + k * _C + o, 16)] = f0 + 1
            return 0

        lax.fori_loop(0, _G, grp, 0)
        ccs = jnp.min(cc)
        for f in range(2):
            pltpu.async_copy(
                tflat.at[ccs].at[idx_v.at[pl.ds(f * 4 * _C, 4 * _C)]],
                rows_v.at[pl.ds(f * 4 * _C, 4 * _C)],
                sems.at[sem_i])

    def drain_blend(c, idx_v, w_v, rows_v, sem_i):
        for f in range(2):
            pltpu.make_async_copy(
                tflat.at[0].at[idx_v.at[pl.ds(f * 4 * _C, 4 * _C)]],
                rows_v.at[pl.ds(f * 4 * _C, 4 * _C)],
                sems.at[sem_i]).wait()
        colbase = 2 * c

        def grp(g, _):
            o = g * 16
            w0 = w_v[pl.ds(o, 16)]
            w1 = w_v[pl.ds(_C + o, 16)]
            u0 = 1.0 - w0
            u1 = 1.0 - w1
            obase = (g * 16 + iota) * 96 + colbase
            for f in range(2):
                rb = f * 4 * _C + o
                e00 = rows_v[pl.ds(rb, 16)]
                e01 = rows_v[pl.ds(rb + _C, 16)]
                e10 = rows_v[pl.ds(rb + 2 * _C, 16)]
                e11 = rows_v[pl.ds(rb + 3 * _C, 16)]
                v0 = e00 * u0 + e10 * w0
                v1 = e01 * u0 + e11 * w0
                plsc.store_scatter(out_v, [obase + f], v0 * u1 + v1 * w1)
            return 0

        lax.fori_loop(0, _G, grp, 0)

    def chunk_body(ch, _):
        base = wid * _PPT + ch * _C
        pltpu.sync_copy(xf.at[pl.ds(base, _C), :], x_v)
        for u in range(3):
            fire(jnp.int32(_NRC + u), idx_vs[u], w_vs[u], rows_vs[u], u)

        lax.fori_loop(0, _NRC, lambda rc, _: (resident(rc), 0)[1], 0)

        def hbm_body(t, _):
            for u in range(4):
                c = _NRC + 4 * t + u
                un = (u + 3) % 4

                @pl.when(c < _NCOMBO)
                def _():
                    drain_blend(c, idx_vs[u], w_vs[u], rows_vs[u], u)

                @pl.when(c + 3 < _NCOMBO)
                def _():
                    fire(c + 3, idx_vs[un], w_vs[un], rows_vs[un], un)
            return 0

        lax.fori_loop(0, (_NCOMBO - _NRC + 3) // 4, hbm_body, 0)
        pltpu.sync_copy(out_v, out.at[pl.ds(base * 96, _C * 96)])
        return 0

    lax.fori_loop(0, _NCHUNK, chunk_body, 0)


@jax.jit
def kernel(x, tables):
    xf = x
    tflat = tables.reshape(3 * _N_LEVELS, _TS * 2)
    invg = jnp.asarray(_INVG_NP)
    icon = jnp.asarray(_ICON_NP)
    iconr = jnp.asarray(_ICONR_NP)
    bidx = jnp.asarray(_BIDX_NP)
    mesh = plsc.VectorSubcoreMesh(core_axis_name="c", subcore_axis_name="s")
    run = pl.kernel(
        _tile_body,
        out_type=jax.ShapeDtypeStruct((_B * 96,), jnp.float32),
        mesh=mesh,
        compiler_params=pltpu.CompilerParams(needs_layout_passes=False,
                                             use_tc_tiling_on_sc=False),
        scratch_types=[
            pltpu.VMEM((_C, 3), jnp.float32),
            pltpu.VMEM((_C * 96,), jnp.float32),
            pltpu.VMEM((_STB_WORDS,), jnp.float32),
            [pltpu.VMEM((2 * 4 * _C,), jnp.int32) for _ in range(4)],
            [pltpu.VMEM((2 * 4 * _C,), jnp.float32) for _ in range(4)],
            [pltpu.VMEM((2 * _C,), jnp.float32) for _ in range(4)],
            pltpu.VMEM((_NCOMBO * 16,), jnp.float32),
            pltpu.VMEM((3 * _NCOMBO * 16,), jnp.int32),
            pltpu.VMEM((3 * _NRC * 16,), jnp.int32),
            pltpu.SemaphoreType.DMA((4,)),
        ],
    )
    return run(xf, tflat, invg, icon, iconr, bidx).reshape(_B, 96)


# speedup vs baseline: 1.1082x; 1.1082x over previous
import functools

import numpy as np
import jax
import jax.numpy as jnp
from jax import lax
from jax.experimental import pallas as pl
from jax.experimental.pallas import tpu as pltpu
from jax.experimental.pallas import tpu_sc as plsc

_N_LEVELS = 16
_TS = 2 ** 19
_MASK = _TS - 1
_PRIME = np.int32(np.uint32(2654435761).astype(np.int64) - (1 << 32))
_PAIRS = ((0, 1), (0, 2), (1, 2))
_NCOMBO = _N_LEVELS * 3
_NRLVL = 8
_NRC = 3 * _NRLVL
_NFC = _NCOMBO - _NRC

_B = 1048576
_NT = 32
_PPT = _B // _NT
_C = 128
_G = _C // 16
_NCHUNK = _PPT // _C
_BSEG = 2048
_FSEG = 2048


def _resolutions():
    growth = np.exp((np.log(512.0) - np.log(16.0)) / (_N_LEVELS - 1))
    return [int(np.floor(16.0 * (growth ** i))) for i in range(_N_LEVELS)]


def _combo_consts():
    invg = np.zeros((_NCOMBO, 16), np.float32)
    icon = np.zeros((3, _NCOMBO, 16), np.int32)
    for i, res in enumerate(_resolutions()):
        for j, (a, b) in enumerate(_PAIRS):
            c = 3 * i + j
            invg[c, :] = np.float32(res / 2.0)
            icon[0, c, :] = j * _N_LEVELS + i
            icon[1, c, :] = a
            icon[2, c, :] = b
    return invg.reshape(-1), icon.reshape(-1)


def _subtable_consts():
    iconr = np.zeros((3, _NRC, 16), np.int32)
    bidx_parts = []
    plan = []
    sb = 0
    boff = 0
    for i, res in enumerate(_resolutions()[:_NRLVL]):
        lo = res // 2
        W = res - lo + 1
        e = np.arange(W * W, dtype=np.int64)
        c0 = lo + e // W
        c1 = lo + e % W
        h = ((c0 ^ (c1 * 2654435761)) & _MASK).astype(np.int64)
        seg = np.empty(2 * W * W, dtype=np.int32)
        seg[0::2] = 2 * h
        seg[1::2] = 2 * h + 1
        pad = (-len(seg)) % 8
        if pad:
            seg = np.concatenate([seg, np.zeros(pad, np.int32)])
        for j in range(3):
            rc = 3 * i + j
            iconr[0, rc, :] = lo
            iconr[1, rc, :] = 2 * W
            iconr[2, rc, :] = sb
            bidx_parts.append(seg)
            for s in range(0, len(seg), _BSEG):
                ln = min(_BSEG, len(seg) - s)
                plan.append((j * _N_LEVELS + i, boff + s, sb + s, ln))
            sb += len(seg)
            boff += len(seg)
    return iconr.reshape(-1), np.concatenate(bidx_parts), plan, sb


def _fine_consts():
    iconf = np.zeros((3, _NFC, 16), np.int32)
    fbidx_parts = []
    plan = []
    sp = 0
    for i in range(_NRLVL, _N_LEVELS):
        res = _resolutions()[i]
        lo = res // 2
        W = res - lo + 1
        e = np.arange(W * W, dtype=np.int64)
        c0 = lo + e // W
        c1 = lo + e % W
        h = ((c0 ^ (c1 * 2654435761)) & _MASK).astype(np.int64)
        seg = np.empty(2 * W * W, dtype=np.int32)
        seg[0::2] = 2 * h
        seg[1::2] = 2 * h + 1
        pad = (-len(seg)) % _FSEG
        if pad:
            seg = np.concatenate([seg, np.zeros(pad, np.int32)])
        for j in range(3):
            fc = 3 * (i - _NRLVL) + j
            iconf[0, fc, :] = lo
            iconf[1, fc, :] = 2 * W
            iconf[2, fc, :] = sp
            fbidx_parts.append(seg)
            plan.append((j * _N_LEVELS + i, sp, len(seg) // _FSEG))
            sp += len(seg)
    return iconf.reshape(-1), np.concatenate(fbidx_parts), plan, sp


_INVG_NP, _ICON_NP = _combo_consts()
_ICONR_NP, _BIDX_NP, _BUILD_PLAN, _STB_WORDS = _subtable_consts()
_ICONF_NP, _FBIDX_NP, _FPLAN, _SPM_WORDS = _fine_consts()
assert 16 * (27408 + 29312) + _SPM_WORDS < 2_000_000, _SPM_WORDS


def _tile_body(xf, tflat, invg, icon, iconr, bidx, fbidx, iconf, out,
               x_v, out_v, stb_v, idx_vs, rows_vs, w_vs,
               invg_v, icon_v, iconr_v, iconf_v, fidx_v, spm, sems):
    sid = lax.axis_index("s")
    wid = sid * 2 + lax.axis_index("c")
    iota = lax.iota(jnp.int32, 16)
    pltpu.sync_copy(invg, invg_v)
    pltpu.sync_copy(icon, icon_v)
    pltpu.sync_copy(iconr, iconr_v)
    pltpu.sync_copy(iconf, iconf_v)

    for row, base, nch in _FPLAN:
        def bchunk(u, _):
            t = u * 16 + sid

            @pl.when(t < nch)
            def _():
                off = base + t * _FSEG
                pltpu.sync_copy(fbidx.at[pl.ds(off, _FSEG)], fidx_v)
                pltpu.async_copy(tflat.at[row].at[fidx_v],
                                 stb_v.at[pl.ds(0, _FSEG)], sems.at[0])
                pltpu.make_async_copy(tflat.at[row].at[fidx_v],
                                      stb_v.at[pl.ds(0, _FSEG)],
                                      sems.at[0]).wait()
                pltpu.sync_copy(stb_v.at[pl.ds(0, _FSEG)],
                                spm.at[pl.ds(off, _FSEG)])
            return 0

        lax.fori_loop(0, (nch + 15) // 16, bchunk, 0)

    for row, bo, so, ln in _BUILD_PLAN:
        pltpu.sync_copy(bidx.at[pl.ds(bo, ln)], fidx_v.at[pl.ds(0, ln)])
        pltpu.async_copy(
            tflat.at[row].at[fidx_v.at[pl.ds(0, ln)]],
            stb_v.at[pl.ds(so, ln)], sems.at[0])
        pltpu.make_async_copy(
            tflat.at[row].at[fidx_v.at[pl.ds(0, ln)]],
            stb_v.at[pl.ds(so, ln)], sems.at[0]).wait()

    plsc.subcore_barrier()

    def loadx(g, conda, condb):
        o = g * 16
        rows = iota + o
        x0 = plsc.load_gather(x_v, [rows, iota * 0])
        x1 = plsc.load_gather(x_v, [rows, iota * 0 + 1])
        x2 = plsc.load_gather(x_v, [rows, iota * 0 + 2])
        xa = jnp.where(conda, x0, x1)
        xb = jnp.where(condb, x1, x2)
        return xa, xb

    def combo_consts(c):
        igv = invg_v[pl.ds(c * 16, 16)]
        av = icon_v[pl.ds((_NCOMBO + c) * 16, 16)]
        bv = icon_v[pl.ds((2 * _NCOMBO + c) * 16, 16)]
        return igv, av == 0, bv == 1

    def resident(rc):
        igv, conda, condb = combo_consts(rc)
        lo = iconr_v[pl.ds(rc * 16, 16)]
        w2 = iconr_v[pl.ds((_NRC + rc) * 16, 16)]
        sb = iconr_v[pl.ds((2 * _NRC + rc) * 16, 16)]
        iam = lo + lax.shift_right_logical(w2, 1) - 2
        colbase = 2 * rc

        def grp(g, _):
            o = g * 16
            xa, xb = loadx(g, conda, condb)
            ta = (xa + 1.0) * igv
            tb = (xb + 1.0) * igv
            ia0 = jnp.minimum(ta.astype(jnp.int32), iam)
            ib0 = jnp.minimum(tb.astype(jnp.int32), iam)
            w0 = ta - ia0.astype(jnp.float32)
            w1 = tb - ib0.astype(jnp.float32)
            u0 = 1.0 - w0
            u1 = 1.0 - w1
            widx = sb + (ia0 - lo) * w2 + lax.shift_left(ib0 - lo, 1)
            g10 = widx + w2
            obase = (g * 16 + iota) * 96 + colbase
            for f in range(2):
                e00 = plsc.load_gather(stb_v, [widx + f])
                e01 = plsc.load_gather(stb_v, [widx + (2 + f)])
                e10 = plsc.load_gather(stb_v, [g10 + f])
                e11 = plsc.load_gather(stb_v, [g10 + (2 + f)])
                v0 = e00 * u0 + e10 * w0
                v1 = e01 * u0 + e11 * w0
                plsc.store_scatter(out_v, [obase + f], v0 * u1 + v1 * w1)
            return 0

        lax.fori_loop(0, _G, grp, 0)

    def fire(c, idx_v, w_v, rows_v, sem_i):
        igv, conda, condb = combo_consts(c)
        fc = c - _NRC
        lo = iconf_v[pl.ds(fc * 16, 16)]
        w2 = iconf_v[pl.ds((_NFC + fc) * 16, 16)]
        sb = iconf_v[pl.ds((2 * _NFC + fc) * 16, 16)]
        iam = lo + lax.shift_right_logical(w2, 1) - 2

        def grp(g, _):
            o = g * 16
            xa, xb = loadx(g, conda, condb)
            ta = (xa + 1.0) * igv
            tb = (xb + 1.0) * igv
            ia0 = jnp.minimum(ta.astype(jnp.int32), iam)
            ib0 = jnp.minimum(tb.astype(jnp.int32), iam)
            w_v[pl.ds(o, 16)] = ta - ia0.astype(jnp.float32)
            w_v[pl.ds(_C + o, 16)] = tb - ib0.astype(jnp.float32)
            base = sb + (ia0 - lo) * w2 + lax.shift_left(ib0 - lo, 1)
            for k, off in enumerate((base, base + 2,
                                     base + w2, base + w2 + 2)):
                idx_v[pl.ds(k * _C + o, 16)] = off
                idx_v[pl.ds(4 * _C + k * _C + o, 16)] = off + 1
            return 0

        lax.fori_loop(0, _G, grp, 0)
        for f in range(2):
            pltpu.async_copy(
                spm.at[idx_v.at[pl.ds(f * 4 * _C, 4 * _C)]],
                rows_v.at[pl.ds(f * 4 * _C, 4 * _C)],
                sems.at[sem_i])

    def drain_blend(c, idx_v, w_v, rows_v, sem_i):
        for f in range(2):
            pltpu.make_async_copy(
                spm.at[idx_v.at[pl.ds(f * 4 * _C, 4 * _C)]],
                rows_v.at[pl.ds(f * 4 * _C, 4 * _C)],
                sems.at[sem_i]).wait()
        colbase = 2 * c

        def grp(g, _):
            o = g * 16
            w0 = w_v[pl.ds(o, 16)]
            w1 = w_v[pl.ds(_C + o, 16)]
            u0 = 1.0 - w0
            u1 = 1.0 - w1
            obase = (g * 16 + iota) * 96 + colbase
            for f in range(2):
                rb = f * 4 * _C + o
                e00 = rows_v[pl.ds(rb, 16)]
                e01 = rows_v[pl.ds(rb + _C, 16)]
                e10 = rows_v[pl.ds(rb + 2 * _C, 16)]
                e11 = rows_v[pl.ds(rb + 3 * _C, 16)]
                v0 = e00 * u0 + e10 * w0
                v1 = e01 * u0 + e11 * w0
                plsc.store_scatter(out_v, [obase + f], v0 * u1 + v1 * w1)
            return 0

        lax.fori_loop(0, _G, grp, 0)

    def chunk_body(ch, _):
        base = wid * _PPT + ch * _C
        pltpu.sync_copy(xf.at[pl.ds(base, _C), :], x_v)
        for u in range(3):
            fire(jnp.int32(_NRC + u), idx_vs[u], w_vs[u], rows_vs[u], u)

        lax.fori_loop(0, _NRC, lambda rc, _: (resident(rc), 0)[1], 0)

        def hbm_body(t, _):
            for u in range(4):
                c = _NRC + 4 * t + u
                un = (u + 3) % 4

                @pl.when(c < _NCOMBO)
                def _():
                    drain_blend(c, idx_vs[u], w_vs[u], rows_vs[u], u)

                @pl.when(c + 3 < _NCOMBO)
                def _():
                    fire(c + 3, idx_vs[un], w_vs[un], rows_vs[un], un)
            return 0

        lax.fori_loop(0, (_NCOMBO - _NRC + 3) // 4, hbm_body, 0)
        pltpu.sync_copy(out_v, out.at[pl.ds(base * 96, _C * 96)])
        return 0

    lax.fori_loop(0, _NCHUNK, chunk_body, 0)


@jax.jit
def kernel(x, tables):
    xf = x
    tflat = tables.reshape(3 * _N_LEVELS, _TS * 2)
    invg = jnp.asarray(_INVG_NP)
    icon = jnp.asarray(_ICON_NP)
    iconr = jnp.asarray(_ICONR_NP)
    bidx = jnp.asarray(_BIDX_NP)
    fbidx = jnp.asarray(_FBIDX_NP)
    iconf = jnp.asarray(_ICONF_NP)
    mesh = plsc.VectorSubcoreMesh(core_axis_name="c", subcore_axis_name="s")
    run = pl.kernel(
        _tile_body,
        out_type=jax.ShapeDtypeStruct((_B * 96,), jnp.float32),
        mesh=mesh,
        compiler_params=pltpu.CompilerParams(needs_layout_passes=False,
                                             use_tc_tiling_on_sc=False),
        scratch_types=[
            pltpu.VMEM((_C, 3), jnp.float32),
            pltpu.VMEM((_C * 96,), jnp.float32),
            pltpu.VMEM((_STB_WORDS,), jnp.float32),
            [pltpu.VMEM((2 * 4 * _C,), jnp.int32) for _ in range(4)],
            [pltpu.VMEM((2 * 4 * _C,), jnp.float32) for _ in range(4)],
            [pltpu.VMEM((2 * _C,), jnp.float32) for _ in range(4)],
            pltpu.VMEM((_NCOMBO * 16,), jnp.float32),
            pltpu.VMEM((3 * _NCOMBO * 16,), jnp.int32),
            pltpu.VMEM((3 * _NRC * 16,), jnp.int32),
            pltpu.VMEM((3 * _NFC * 16,), jnp.int32),
            pltpu.VMEM((_FSEG,), jnp.int32),
            pltpu.VMEM_SHARED((_SPM_WORDS,), jnp.float32),
            pltpu.SemaphoreType.DMA((4,)),
        ],
    )
    return run(xf, tflat, invg, icon, iconr, bidx, fbidx,
               iconf).reshape(_B, 96)
